# baseline (speedup 1.0000x reference)
import dataclasses

import jax
import jax.numpy as jnp
from jax import lax
from jax.experimental import pallas as pl
from jax.experimental.pallas import tpu as pltpu
from jax.experimental.pallas import tpu_sc as plsc

_G = 128
_NC, _NS = 2, 16
_NW = _NC * _NS


def kernel(indices, embed_table):
    batch, seq = indices.shape
    n = batch * seq
    vocab, embed_dim = embed_table.shape
    pack = 128 // embed_dim
    table128 = embed_table.reshape(vocab // pack, 128)
    idx_flat = indices.reshape(n).astype(jnp.int32)

    per_w = n // _NW
    groups = per_w // _G

    mesh = plsc.VectorSubcoreMesh(core_axis_name="c", subcore_axis_name="s")
    cp = pltpu.CompilerParams()
    if "needs_layout_passes" in pltpu.CompilerParams.__dataclass_fields__:
        cp = dataclasses.replace(cp, needs_layout_passes=False)

    @pl.kernel(
        out_type=jax.ShapeDtypeStruct((n, embed_dim), embed_table.dtype),
        mesh=mesh,
        compiler_params=cp,
        scratch_types=[
            pltpu.VMEM((_G,), jnp.int32),
            pltpu.VMEM((_G,), jnp.int32),
            pltpu.VMEM((_G,), jnp.int32),
            pltpu.VMEM((_G, 128), jnp.float32),
            pltpu.VMEM((_G, embed_dim), jnp.float32),
            pltpu.SemaphoreType.DMA,
        ],
    )
    def gather_kernel(table_hbm, idx_hbm, out_hbm,
                      idx_v, hi_v, off_v, rows_v, out_v, sem):
        wid = lax.axis_index("s") * _NC + lax.axis_index("c")
        base = wid * per_w
        iota16 = lax.iota(jnp.int32, 16)

        @pl.loop(0, groups)
        def _(g):
            g0 = base + g * _G
            pltpu.sync_copy(idx_hbm.at[pl.ds(g0, _G)], idx_v)
            for k in range(_G // 16):
                sl = pl.ds(k * 16, 16)
                v = idx_v[sl]
                hi_v[sl] = jax.lax.shift_right_logical(v, 2)
                off_v[sl] = (v & (pack - 1)) * embed_dim
            pltpu.async_copy(table_hbm.at[hi_v], rows_v, sem).wait()

            for c in range(_G // 16):
                rows16 = iota16 + c * 16
                offv = off_v[pl.ds(c * 16, 16)]
                for l in range(embed_dim):
                    vals = plsc.load_gather(rows_v, [rows16, offv + l])
                    plsc.store_scatter(
                        out_v, [rows16, jnp.full((16,), l, jnp.int32)], vals)

            pltpu.sync_copy(out_v, out_hbm.at[pl.ds(g0, _G)])

    out = gather_kernel(table128, idx_flat)
    return out.reshape(batch, seq, embed_dim)


# speedup vs baseline: 1.2009x; 1.2009x over previous
import dataclasses

import jax
import jax.numpy as jnp
from jax import lax
from jax.experimental import pallas as pl
from jax.experimental.pallas import tpu as pltpu
from jax.experimental.pallas import tpu_sc as plsc

_NC, _NS = 2, 16
_NW = _NC * _NS


def kernel(indices, embed_table):
    batch, seq = indices.shape
    vocab, embed_dim = embed_table.shape
    pack = 128 // embed_dim
    table128 = embed_table.reshape(vocab // pack, 128)
    idx = indices.reshape(batch * seq).astype(jnp.int32)

    bpw = batch // _NW
    nch = -(-seq // 16)
    spad = nch * 16
    s0 = (seq // 128) * 128
    s1 = seq - s0

    mesh = plsc.VectorSubcoreMesh(core_axis_name="c", subcore_axis_name="s")
    cp = pltpu.CompilerParams()
    if "needs_layout_passes" in pltpu.CompilerParams.__dataclass_fields__:
        cp = dataclasses.replace(cp, needs_layout_passes=False)

    @pl.kernel(
        out_type=jax.ShapeDtypeStruct((batch, seq, embed_dim),
                                      embed_table.dtype),
        mesh=mesh,
        compiler_params=cp,
        scratch_types=(
            [pltpu.VMEM((spad,), jnp.int32)] * 2
            + [pltpu.VMEM((spad,), jnp.int32)] * 2
            + [pltpu.VMEM((spad,), jnp.int32)] * 2
            + [pltpu.VMEM((spad, 128), jnp.float32)] * 2
            + [pltpu.VMEM((spad, embed_dim), jnp.float32)] * 2
            + [pltpu.SemaphoreType.DMA] * 6
        ),
    )
    def gather_kernel(table_hbm, idx_hbm, out_hbm,
                      idx_v0, idx_v1, hi_v0, hi_v1, off_v0, off_v1,
                      rows_v0, rows_v1, out_v0, out_v1,
                      isem0, isem1, gsem0, gsem1, wsem0, wsem1):
        idx_v = (idx_v0, idx_v1)
        hi_v = (hi_v0, hi_v1)
        off_v = (off_v0, off_v1)
        rows_v = (rows_v0, rows_v1)
        out_v = (out_v0, out_v1)
        isem = (isem0, isem1)
        gsem = (gsem0, gsem1)
        wsem = (wsem0, wsem1)
        wid = lax.axis_index("s") * _NC + lax.axis_index("c")
        b0 = wid * bpw
        iota16 = lax.iota(jnp.int32, 16)

        def fire_idx(gb, j):
            pltpu.async_copy(idx_hbm.at[pl.ds(gb * seq, seq)],
                             idx_v[j].at[pl.ds(0, seq)], isem[j])

        def wait_idx(j):
            pltpu.make_async_copy(idx_hbm.at[pl.ds(0, seq)],
                                  idx_v[j].at[pl.ds(0, seq)],
                                  isem[j]).wait()

        def comp(j):
            for k in range(nch):
                sl = pl.ds(k * 16, 16)
                v = idx_v[j][sl]
                hi_v[j][sl] = jax.lax.shift_right_logical(v, 2)
                off_v[j][sl] = (v & (pack - 1)) * embed_dim

        def fire_gather(j):
            pltpu.async_copy(table_hbm.at[hi_v[j].at[pl.ds(0, s0)]],
                             rows_v[j].at[pl.ds(0, s0)], gsem[j])
            pltpu.async_copy(table_hbm.at[hi_v[j].at[pl.ds(s0, s1)]],
                             rows_v[j].at[pl.ds(s0, s1)], gsem[j])

        def wait_gather(j):
            pltpu.make_async_copy(table_hbm.at[hi_v[j].at[pl.ds(0, s0)]],
                                  rows_v[j].at[pl.ds(0, s0)],
                                  gsem[j]).wait()
            pltpu.make_async_copy(table_hbm.at[hi_v[j].at[pl.ds(s0, s1)]],
                                  rows_v[j].at[pl.ds(s0, s1)],
                                  gsem[j]).wait()

        def extract(j):
            for c in range(nch):
                rows16 = iota16 + c * 16
                offv = off_v[j][pl.ds(c * 16, 16)]
                for l in range(embed_dim):
                    vals = plsc.load_gather(rows_v[j],
                                            [rows16, offv + l])
                    plsc.store_scatter(out_v[j],
                                       [rows16, jnp.full((16,), l, jnp.int32)],
                                       vals)

        def fire_wb(gb, j):
            pltpu.async_copy(out_v[j].at[pl.ds(0, seq)],
                             out_hbm.at[gb], wsem[j])

        def wait_wb(j):
            pltpu.make_async_copy(out_v[j].at[pl.ds(0, seq)],
                                  out_hbm.at[0], wsem[j]).wait()

        fire_idx(b0, 0)
        wait_idx(0)
        comp(0)
        fire_gather(0)
        fire_idx(b0 + 1, 1)

        @pl.loop(0, bpw // 2 - 1)
        def _(k):
            b = k * 2
            wait_idx(1)
            comp(1)
            fire_gather(1)
            fire_idx(b0 + b + 2, 0)
            wait_gather(0)

            @pl.when(k > 0)
            def _():
                wait_wb(0)

            extract(0)
            fire_wb(b0 + b, 0)
            wait_idx(0)
            comp(0)
            fire_gather(0)
            fire_idx(b0 + b + 3, 1)
            wait_gather(1)

            @pl.when(k > 0)
            def _():
                wait_wb(1)

            extract(1)
            fire_wb(b0 + b + 1, 1)

        wait_idx(1)
        comp(1)
        fire_gather(1)
        wait_gather(0)
        wait_wb(0)
        extract(0)
        fire_wb(b0 + bpw - 2, 0)
        wait_gather(1)
        wait_wb(1)
        extract(1)
        fire_wb(b0 + bpw - 1, 1)
        wait_wb(0)
        wait_wb(1)

    out = gather_kernel(table128, idx)
    return out
